# Initial kernel scaffold; baseline (speedup 1.0000x reference)
import jax
import jax.numpy as jnp
from jax.experimental import pallas as pl


def kernel(x, position, table):
    raise NotImplementedError("write your pallas kernel here")


# speedup vs baseline: 1.6384x; 1.6384x over previous
import functools
import math

import jax
import jax.numpy as jnp
from jax import lax
from jax.experimental import pallas as pl
from jax.experimental.pallas import tpu as pltpu
from jax.experimental.pallas import tpu_sc as plsc

VOCAB = 1000
D = 128
N = 1024 * 200
LANES = 16

NUM_CORES = 2
NUM_SUBCORES = 16
NW = NUM_CORES * NUM_SUBCORES

CHUNK = 128
ROWS_PER_W = N // NW
CHUNKS_PER_W = ROWS_PER_W // CHUNK

SCALE = math.sqrt(D)


def _sc_body(table_hbm, idx_hbm, pos_hbm, out_hbm, idx_v, rows_v, pos_v, sem):
    wid = lax.axis_index("s") * NUM_CORES + lax.axis_index("c")
    base = wid * ROWS_PER_W

    def chunk_body(c, carry):
        start = base + c * CHUNK
        pltpu.sync_copy(idx_hbm.at[pl.ds(start, CHUNK)], idx_v)
        gather = pltpu.make_async_copy(table_hbm.at[idx_v], rows_v, sem)
        gather.start()
        pltpu.sync_copy(pos_hbm.at[pl.ds(start, CHUNK), :], pos_v)
        gather.wait()

        def row_body(i, carry2):
            for j in range(D // LANES):
                sl = pl.ds(j * LANES, LANES)
                pos_v[i, sl] = rows_v[i, sl] * SCALE + pos_v[i, sl]
            return carry2

        lax.fori_loop(0, CHUNK, row_body, 0, unroll=2)

        pltpu.sync_copy(pos_v, out_hbm.at[pl.ds(start, CHUNK), :])
        return carry

    lax.fori_loop(0, CHUNKS_PER_W, chunk_body, 0)


@jax.jit
def _sc_embed(x_flat, position_flat, table):
    mesh = plsc.VectorSubcoreMesh(core_axis_name="c", subcore_axis_name="s")
    kern = functools.partial(
        pl.kernel,
        mesh=mesh,
        out_type=jax.ShapeDtypeStruct((N, D), jnp.float32),
        scratch_types=[
            pltpu.VMEM((CHUNK,), jnp.int32),
            pltpu.VMEM((CHUNK, D), jnp.float32),
            pltpu.VMEM((CHUNK, D), jnp.float32),
            pltpu.SemaphoreType.DMA,
        ],
    )(_sc_body)
    return kern(table, x_flat, position_flat)


def kernel(x, position, table):
    x_flat = x.reshape(N)
    pos_flat = position.reshape(N, D)
    out = _sc_embed(x_flat, pos_flat, table)
    return out.reshape(position.shape)


# speedup vs baseline: 3.2966x; 2.0121x over previous
import functools
import math

import jax
import jax.numpy as jnp
from jax import lax
from jax.experimental import pallas as pl
from jax.experimental.pallas import tpu as pltpu
from jax.experimental.pallas import tpu_sc as plsc

VOCAB = 1000
D = 128
N = 1024 * 200
LANES = 16

NUM_CORES = 2
NUM_SUBCORES = 16
NW = NUM_CORES * NUM_SUBCORES

CHUNK = 128
ROWS_PER_W = N // NW
CHUNKS_PER_W = ROWS_PER_W // CHUNK
NBUF = 2

SCALE = math.sqrt(D)


def _sc_body(table_hbm, idx_hbm, pos_hbm, out_hbm,
             idx_v, rows_v, pos_v, out_v,
             gsem0, gsem1, psem0, psem1, osem0, osem1):
    wid = lax.axis_index("s") * NUM_CORES + lax.axis_index("c")
    base = pl.multiple_of(wid * ROWS_PER_W, CHUNK)
    sems = [(gsem0, psem0, osem0), (gsem1, psem1, osem1)]

    pltpu.sync_copy(idx_hbm.at[pl.ds(base, ROWS_PER_W)], idx_v)

    def in_copies(c, b):
        start = pl.multiple_of(base + c * CHUNK, CHUNK)
        idx_sl = idx_v.at[pl.ds(pl.multiple_of(c * CHUNK, CHUNK), CHUNK)]
        g = pltpu.make_async_copy(table_hbm.at[idx_sl], rows_v.at[b],
                                  sems[b][0])
        p = pltpu.make_async_copy(pos_hbm.at[pl.ds(start, CHUNK), :],
                                  pos_v.at[b], sems[b][1])
        return g, p

    def out_copy(c, b):
        start = pl.multiple_of(base + c * CHUNK, CHUNK)
        return pltpu.make_async_copy(out_v.at[b],
                                     out_hbm.at[pl.ds(start, CHUNK), :],
                                     sems[b][2])

    def compute(b):
        def row_body(i, carry):
            for j in range(D // LANES):
                sl = pl.ds(j * LANES, LANES)
                out_v[b, i, sl] = rows_v[b, i, sl] * SCALE + pos_v[b, i, sl]
            return carry
        lax.fori_loop(0, CHUNK, row_body, 0, unroll=2)

    for b in range(NBUF):
        g, p = in_copies(b, b)
        g.start()
        p.start()

    for b in range(NBUF):
        g, p = in_copies(b, b)
        g.wait()
        p.wait()
        compute(b)
        out_copy(b, b).start()
        g2, p2 = in_copies(b + NBUF, b)
        g2.start()
        p2.start()

    def steady(i, carry):
        for b in range(NBUF):
            c = i * NBUF + b
            g, p = in_copies(c, b)
            g.wait()
            p.wait()
            out_copy(c - NBUF, b).wait()
            compute(b)
            out_copy(c, b).start()
            g2, p2 = in_copies(c + NBUF, b)
            g2.start()
            p2.start()
        return carry

    lax.fori_loop(1, CHUNKS_PER_W // NBUF - 1, steady, 0)

    for b in range(NBUF):
        c = CHUNKS_PER_W - NBUF + b
        g, p = in_copies(c, b)
        g.wait()
        p.wait()
        out_copy(c - NBUF, b).wait()
        compute(b)
        out_copy(c, b).start()

    for b in range(NBUF):
        out_copy(CHUNKS_PER_W - NBUF + b, b).wait()


@jax.jit
def _sc_embed(x_flat, position_flat, table):
    mesh = plsc.VectorSubcoreMesh(core_axis_name="c", subcore_axis_name="s")
    kern = functools.partial(
        pl.kernel,
        mesh=mesh,
        out_type=jax.ShapeDtypeStruct((N, D), jnp.float32),
        scratch_types=[
            pltpu.VMEM((ROWS_PER_W,), jnp.int32),
            pltpu.VMEM((NBUF, CHUNK, D), jnp.float32),
            pltpu.VMEM((NBUF, CHUNK, D), jnp.float32),
            pltpu.VMEM((NBUF, CHUNK, D), jnp.float32),
            pltpu.SemaphoreType.DMA,
            pltpu.SemaphoreType.DMA,
            pltpu.SemaphoreType.DMA,
            pltpu.SemaphoreType.DMA,
            pltpu.SemaphoreType.DMA,
            pltpu.SemaphoreType.DMA,
        ],
    )(_sc_body)
    return kern(table, x_flat, position_flat)


def kernel(x, position, table):
    x_flat = x.reshape(N)
    pos_flat = position.reshape(N, D)
    out = _sc_embed(x_flat, pos_flat, table)
    return out.reshape(position.shape)


# speedup vs baseline: 7.5070x; 2.2772x over previous
import functools
import math

import jax
import jax.numpy as jnp
from jax import lax
from jax.experimental import pallas as pl
from jax.experimental.pallas import tpu as pltpu
from jax.experimental.pallas import tpu_sc as plsc

VOCAB = 1000
D = 128
N = 1024 * 200
LANES = 16

NUM_CORES = 2
NUM_SUBCORES = 16
NW = NUM_CORES * NUM_SUBCORES

CHUNK = 128
ROWS_PER_W = N // NW
CHUNKS_PER_W = ROWS_PER_W // CHUNK
NBUF = 2

SCALE = math.sqrt(D)


def _sc_body(table_hbm, idx_hbm, pos_hbm, out_hbm,
             table_sh, idx_v, rows_v, pos_v, out_v,
             tsem, gsem0, gsem1, psem0, psem1, osem0, osem1):
    sid = lax.axis_index("s")
    wid = sid * NUM_CORES + lax.axis_index("c")
    base = pl.multiple_of(wid * ROWS_PER_W, CHUNK)
    sems = [(gsem0, psem0, osem0), (gsem1, psem1, osem1)]

    @pl.when(sid == 0)
    def _():
        pltpu.make_async_copy(table_hbm, table_sh, tsem).start()

    pltpu.sync_copy(idx_hbm.at[pl.ds(base, ROWS_PER_W)], idx_v)

    @pl.when(sid == 0)
    def _():
        pltpu.make_async_copy(table_hbm, table_sh, tsem).wait()

    plsc.subcore_barrier()

    def in_copies(c, b):
        start = pl.multiple_of(base + c * CHUNK, CHUNK)
        idx_sl = idx_v.at[pl.ds(pl.multiple_of(c * CHUNK, CHUNK), CHUNK)]
        g = pltpu.make_async_copy(table_sh.at[idx_sl], rows_v.at[b],
                                  sems[b][0])
        p = pltpu.make_async_copy(pos_hbm.at[pl.ds(start, CHUNK), :],
                                  pos_v.at[b], sems[b][1])
        return g, p

    def out_copy(c, b):
        start = pl.multiple_of(base + c * CHUNK, CHUNK)
        return pltpu.make_async_copy(out_v.at[b],
                                     out_hbm.at[pl.ds(start, CHUNK), :],
                                     sems[b][2])

    def compute(b):
        @plsc.parallel_loop(0, CHUNK, 1, unroll=4)
        def row_body(i):
            for j in range(D // LANES):
                sl = pl.ds(j * LANES, LANES)
                out_v[b, i, sl] = rows_v[b, i, sl] * SCALE + pos_v[b, i, sl]

    for b in range(NBUF):
        g, p = in_copies(b, b)
        g.start()
        p.start()

    for b in range(NBUF):
        g, p = in_copies(b, b)
        g.wait()
        p.wait()
        compute(b)
        out_copy(b, b).start()
        g2, p2 = in_copies(b + NBUF, b)
        g2.start()
        p2.start()

    def steady(i, carry):
        for b in range(NBUF):
            c = i * NBUF + b
            g, p = in_copies(c, b)
            g.wait()
            p.wait()
            out_copy(c - NBUF, b).wait()
            compute(b)
            out_copy(c, b).start()
            g2, p2 = in_copies(c + NBUF, b)
            g2.start()
            p2.start()
        return carry

    lax.fori_loop(1, CHUNKS_PER_W // NBUF - 1, steady, 0)

    for b in range(NBUF):
        c = CHUNKS_PER_W - NBUF + b
        g, p = in_copies(c, b)
        g.wait()
        p.wait()
        out_copy(c - NBUF, b).wait()
        compute(b)
        out_copy(c, b).start()

    for b in range(NBUF):
        out_copy(CHUNKS_PER_W - NBUF + b, b).wait()


@jax.jit
def _sc_embed(x_flat, position_flat, table):
    mesh = plsc.VectorSubcoreMesh(core_axis_name="c", subcore_axis_name="s")
    kern = functools.partial(
        pl.kernel,
        mesh=mesh,
        out_type=jax.ShapeDtypeStruct((N, D), jnp.float32),
        scratch_types=[
            pltpu.VMEM_SHARED((VOCAB, D), jnp.float32),
            pltpu.VMEM((ROWS_PER_W,), jnp.int32),
            pltpu.VMEM((NBUF, CHUNK, D), jnp.float32),
            pltpu.VMEM((NBUF, CHUNK, D), jnp.float32),
            pltpu.VMEM((NBUF, CHUNK, D), jnp.float32),
            pltpu.SemaphoreType.DMA,
            pltpu.SemaphoreType.DMA,
            pltpu.SemaphoreType.DMA,
            pltpu.SemaphoreType.DMA,
            pltpu.SemaphoreType.DMA,
            pltpu.SemaphoreType.DMA,
            pltpu.SemaphoreType.DMA,
        ],
    )(_sc_body)
    return kern(table, x_flat, position_flat)


def kernel(x, position, table):
    x_flat = x.reshape(N)
    pos_flat = position.reshape(N, D)
    out = _sc_embed(x_flat, pos_flat, table)
    return out.reshape(position.shape)
